# baseline (speedup 1.0000x reference)
import jax
import jax.numpy as jnp
from jax import lax
from jax.experimental import pallas as pl
from jax.experimental.pallas import tpu as pltpu
from jax.experimental.pallas import tpu_sc as plsc

_NC = 2
_NS = 16
_NW = _NC * _NS
_B = 16384
_BPW = _B // _NW
_D = 16
_CH = 128
_NCH = _BPW // _CH


def _dice_body(uid_hbm, iid_hbm, uint_hbm, iint_hbm, upop_hbm, ipop_hbm,
               out_hbm, uid_v, iid_v, ui_v, ii_v, up_v, ip_v, out_v, sem):
    wid = lax.axis_index("s") * _NC + lax.axis_index("c")
    rbase = wid * _NCH

    pltpu.sync_copy(uid_hbm.at[pl.ds(rbase, _NCH)], uid_v)
    pltpu.sync_copy(iid_hbm.at[pl.ds(rbase, _NCH)], iid_v)

    copies = []
    for j in range(_NCH):
        sl = pl.ds(j * _CH, _CH)
        copies.append(pltpu.async_copy(uint_hbm.at[uid_v.at[j]], ui_v.at[sl], sem))
        copies.append(pltpu.async_copy(iint_hbm.at[iid_v.at[j]], ii_v.at[sl], sem))
        copies.append(pltpu.async_copy(upop_hbm.at[uid_v.at[j]], up_v.at[sl], sem))
        copies.append(pltpu.async_copy(ipop_hbm.at[iid_v.at[j]], ip_v.at[sl], sem))
    for c in copies:
        c.wait()

    def block_body(b, carry):
        rows = b * 16 + lax.iota(jnp.int32, 16)
        acc = jnp.zeros((16,), jnp.float32)
        for d in range(_D):
            col = jnp.full((16,), d, jnp.int32)
            acc += plsc.load_gather(ui_v, [rows, col]) * plsc.load_gather(ii_v, [rows, col])
            acc += plsc.load_gather(up_v, [rows, col]) * plsc.load_gather(ip_v, [rows, col])
        out_v[pl.ds(b * 16, 16)] = acc
        return carry

    lax.fori_loop(0, _BPW // 16, block_body, 0)

    pltpu.sync_copy(out_v, out_hbm.at[pl.ds(wid * _BPW, _BPW)])


def kernel(uid_batch, iid_batch, user_int, item_int, user_pop, item_pop):
    uid2 = uid_batch.astype(jnp.int32).reshape(_NW * _NCH, _CH)
    iid2 = iid_batch.astype(jnp.int32).reshape(_NW * _NCH, _CH)
    f = pl.kernel(
        _dice_body,
        mesh=plsc.VectorSubcoreMesh(core_axis_name="c", subcore_axis_name="s"),
        out_type=jax.ShapeDtypeStruct((_B,), jnp.float32),
        scratch_types=[
            pltpu.VMEM((_NCH, _CH), jnp.int32),
            pltpu.VMEM((_NCH, _CH), jnp.int32),
            pltpu.VMEM((_BPW, _D), jnp.float32),
            pltpu.VMEM((_BPW, _D), jnp.float32),
            pltpu.VMEM((_BPW, _D), jnp.float32),
            pltpu.VMEM((_BPW, _D), jnp.float32),
            pltpu.VMEM((_BPW,), jnp.float32),
            pltpu.SemaphoreType.DMA,
        ],
        compiler_params=pltpu.CompilerParams(
            needs_layout_passes=False, use_tc_tiling_on_sc=False),
    )
    return f(uid2, iid2, user_int, item_int, user_pop, item_pop)


# speedup vs baseline: 1.0011x; 1.0011x over previous
import jax
import jax.numpy as jnp
from jax import lax
from jax.experimental import pallas as pl
from jax.experimental.pallas import tpu as pltpu
from jax.experimental.pallas import tpu_sc as plsc

_NC = 2
_NS = 16
_NW = _NC * _NS
_B = 16384
_BPW = _B // _NW
_D = 16
_CH = 128
_NCH = _BPW // _CH


def _dice_body(uid_hbm, iid_hbm, uint_hbm, iint_hbm, upop_hbm, ipop_hbm,
               out_hbm, uid_v, iid_v, ui_v, ii_v, up_v, ip_v, out_v, sem):
    wid = lax.axis_index("s") * _NC + lax.axis_index("c")
    base = wid * _BPW

    pltpu.sync_copy(uid_hbm.at[pl.ds(base, _BPW)], uid_v)
    pltpu.sync_copy(iid_hbm.at[pl.ds(base, _BPW)], iid_v)

    copies = []
    for j in range(_NCH):
        isl = pl.ds(j * _CH, _CH)
        copies.append(pltpu.async_copy(uint_hbm.at[uid_v.at[isl]], ui_v.at[isl], sem))
        copies.append(pltpu.async_copy(iint_hbm.at[iid_v.at[isl]], ii_v.at[isl], sem))
        copies.append(pltpu.async_copy(upop_hbm.at[uid_v.at[isl]], up_v.at[isl], sem))
        copies.append(pltpu.async_copy(ipop_hbm.at[iid_v.at[isl]], ip_v.at[isl], sem))
    for c in copies:
        c.wait()

    def block_body(b, carry):
        rows = b * 16 + lax.iota(jnp.int32, 16)
        acc = jnp.zeros((16,), jnp.float32)
        for d in range(_D):
            col = jnp.full((16,), d, jnp.int32)
            acc += plsc.load_gather(ui_v, [rows, col]) * plsc.load_gather(ii_v, [rows, col])
            acc += plsc.load_gather(up_v, [rows, col]) * plsc.load_gather(ip_v, [rows, col])
        out_v[pl.ds(b * 16, 16)] = acc
        return carry

    lax.fori_loop(0, _BPW // 16, block_body, 0)

    pltpu.sync_copy(out_v, out_hbm.at[pl.ds(base, _BPW)])


def kernel(uid_batch, iid_batch, user_int, item_int, user_pop, item_pop):
    f = pl.kernel(
        _dice_body,
        mesh=plsc.VectorSubcoreMesh(core_axis_name="c", subcore_axis_name="s"),
        out_type=jax.ShapeDtypeStruct((_B,), jnp.float32),
        scratch_types=[
            pltpu.VMEM((_BPW,), jnp.int32),
            pltpu.VMEM((_BPW,), jnp.int32),
            pltpu.VMEM((_BPW, _D), jnp.float32),
            pltpu.VMEM((_BPW, _D), jnp.float32),
            pltpu.VMEM((_BPW, _D), jnp.float32),
            pltpu.VMEM((_BPW, _D), jnp.float32),
            pltpu.VMEM((_BPW,), jnp.float32),
            pltpu.SemaphoreType.DMA,
        ],
        compiler_params=pltpu.CompilerParams(
            needs_layout_passes=False, use_tc_tiling_on_sc=False),
    )
    return f(uid_batch.astype(jnp.int32), iid_batch.astype(jnp.int32),
             user_int, item_int, user_pop, item_pop)
